# Initial kernel scaffold; baseline (speedup 1.0000x reference)
import jax
import jax.numpy as jnp
from jax.experimental import pallas as pl


def kernel(X, adj_mat, W_rel0, b_rel0, W_root0, W_rel1, b_rel1, W_root1, W_rel2, b_rel2, W_root2):
    raise NotImplementedError("write your pallas kernel here")


# speedup vs baseline: 1.3526x; 1.3526x over previous
import functools

import jax
import jax.numpy as jnp
from jax.experimental import pallas as pl
from jax.experimental.pallas import tpu as pltpu

N = 4096
D = 128
H = 64
BLK = 512
NB = N // BLK


def _gnn_kernel(x_ref, adj_ref, wr0, br0, wo0, wr1, br1, wo1, wr2, br2, wo2,
                out_ref, adj_bf, h_s, g_s):
    l = pl.program_id(0)
    i = pl.program_id(1)

    @pl.when(jnp.logical_and(l == 0, i == 0))
    def _():
        g = jax.lax.dot_general(x_ref[...], wr0[...],
                                (((1,), (1,)), ((), ())),
                                preferred_element_type=jnp.float32)
        g_s[...] = g.astype(jnp.bfloat16)

    @pl.when(jnp.logical_and(l > 0, i == 0))
    def _():
        wr = jnp.where(l == 1, wr1[...], wr2[...])
        g = jax.lax.dot_general(h_s[...], wr,
                                (((1,), (1,)), ((), ())),
                                preferred_element_type=jnp.float32)
        g_s[...] = g.astype(jnp.bfloat16)

    @pl.when(l == 0)
    def _():
        a = adj_ref[...].astype(jnp.bfloat16)
        adj_bf[i] = a
        agg = jax.lax.dot_general(a, g_s[...],
                                  (((0,), (0,)), ((), ())),
                                  preferred_element_type=jnp.float32)
        x_blk = x_ref[pl.ds(i * BLK, BLK), :]
        root = jax.lax.dot_general(x_blk, wo0[...],
                                   (((1,), (1,)), ((), ())),
                                   preferred_element_type=jnp.float32)
        res = jnp.maximum(agg + root + br0[...], 0.0)
        out_ref[...] = res
        h_s[pl.ds(i * BLK, BLK), :] = res

    @pl.when(l > 0)
    def _():
        a = adj_bf[i]
        agg = jax.lax.dot_general(a, g_s[...],
                                  (((0,), (0,)), ((), ())),
                                  preferred_element_type=jnp.float32)
        wo = jnp.where(l == 1, wo1[...], wo2[...])
        br = jnp.where(l == 1, br1[...], br2[...])
        h_blk = h_s[pl.ds(i * BLK, BLK), :]
        root = jax.lax.dot_general(h_blk, wo,
                                   (((1,), (1,)), ((), ())),
                                   preferred_element_type=jnp.float32)
        res = jnp.maximum(agg + root + br, 0.0)
        out_ref[...] = res
        h_s[pl.ds(i * BLK, BLK), :] = res


@functools.partial(jax.jit, static_argnames=("interpret",))
def _run(X, adj_mat, W_rel0, b_rel0, W_root0, W_rel1, b_rel1, W_root1,
         W_rel2, b_rel2, W_root2, interpret=False):
    b0 = b_rel0.reshape(1, H)
    b1 = b_rel1.reshape(1, H)
    b2 = b_rel2.reshape(1, H)
    full = lambda shape: pl.BlockSpec(shape, lambda l, i: (0,) * len(shape))
    return pl.pallas_call(
        _gnn_kernel,
        grid=(3, NB),
        in_specs=[
            full((N, D)),
            pl.BlockSpec((N, BLK),
                         lambda l, i: (0, jnp.where(l == 0, i, 0))),
            full((H, D)), full((1, H)), full((H, D)),
            full((H, H)), full((1, H)), full((H, H)),
            full((H, H)), full((1, H)), full((H, H)),
        ],
        out_specs=pl.BlockSpec((BLK, H), lambda l, i: (i, 0)),
        out_shape=jax.ShapeDtypeStruct((N, H), jnp.float32),
        scratch_shapes=[
            pltpu.VMEM((NB, N, BLK), jnp.bfloat16),
            pltpu.VMEM((N, H), jnp.float32),
            pltpu.VMEM((N, H), jnp.bfloat16),
        ],
        interpret=interpret,
    )(X, adj_mat, W_rel0, b0, W_root0, W_rel1, b1, W_root1, W_rel2, b2, W_root2)


def kernel(X, adj_mat, W_rel0, b_rel0, W_root0, W_rel1, b_rel1, W_root1,
           W_rel2, b_rel2, W_root2):
    return _run(X, adj_mat, W_rel0, b_rel0, W_root0, W_rel1, b_rel1, W_root1,
                W_rel2, b_rel2, W_root2)
